# baseline (speedup 1.0000x reference)
import functools

import jax
import jax.numpy as jnp
from jax import lax
from jax.experimental import pallas as pl
from jax.experimental.pallas import tpu as pltpu
from jax.experimental.pallas import tpu_sc as plsc

NC = 2
NS = 16
CB = 32
NBUF = 6


def _mesh():
    return plsc.VectorSubcoreMesh(core_axis_name="c", subcore_axis_name="s",
                                  num_cores=NC, num_subcores=NS)


def _rows_per_tile(n):
    return ((n + NS - 1) // NS + CB - 1) // CB * CB


def _fill(ref, value, dtype):
    (n,) = ref.shape

    @pl.loop(0, n // 16)
    def _(j):
        ref[pl.ds(j * 16, 16)] = jnp.full((16,), value, dtype)


def _fill2(ref, value, dtype):
    rows, cols = ref.shape

    @pl.loop(0, rows)
    def _(r):
        @pl.loop(0, cols // 16)
        def _(j):
            ref[r, pl.ds(j * 16, 16)] = jnp.full((16,), value, dtype)


def _make_degree_kernel(nch, npad):
    rpt = npad // NS

    @functools.partial(
        pl.kernel,
        out_type=jax.ShapeDtypeStruct((NC, npad), jnp.float32),
        mesh=_mesh(),
        scratch_types=[
            pltpu.VMEM((nch, CB), jnp.int32),
            pltpu.VMEM((CB,), jnp.float32),
            pltpu.VMEM((CB,), jnp.float32),
            pltpu.VMEM_SHARED((npad,), jnp.float32),
            pltpu.SemaphoreType.DMA,
        ],
    )
    def deg_kernel(dst_hbm, out_hbm, didx_v, ones_v, zero_v, acc_sh, sem):
        c = lax.axis_index("c")
        s = lax.axis_index("s")
        wid = c * NS + s
        pltpu.sync_copy(dst_hbm.at[wid], didx_v)
        _fill(ones_v, 1.0, jnp.float32)
        _fill(zero_v, 0.0, jnp.float32)
        for k in range(rpt // CB):
            pltpu.sync_copy(zero_v, acc_sh.at[pl.ds(s * rpt + k * CB, CB)])
        plsc.subcore_barrier()

        @pl.loop(0, nch)
        def _(i):
            pltpu.async_copy(ones_v, acc_sh.at[didx_v.at[i]], sem, add=True)

        @pl.loop(0, nch)
        def _(i):
            pltpu.make_async_copy(ones_v, acc_sh.at[didx_v.at[i]], sem).wait()

        plsc.subcore_barrier()
        pltpu.sync_copy(acc_sh.at[pl.ds(s * rpt, rpt)],
                        out_hbm.at[c, pl.ds(s * rpt, rpt)])

    return deg_kernel


def _make_agg_kernel(nch, npad, d):
    assert nch % NBUF == 0 and nch // NBUF >= 2
    rpt = npad // NS

    @functools.partial(
        pl.kernel,
        out_type=jax.ShapeDtypeStruct((NC, npad, d), jnp.float32),
        mesh=_mesh(),
        compiler_params=pltpu.CompilerParams(use_tc_tiling_on_sc=False),
        scratch_types=(
            [pltpu.VMEM((nch, CB), jnp.int32)] * 2
            + [pltpu.VMEM_SHARED((npad, d), jnp.float32)]
            + [pltpu.VMEM((CB, d), jnp.float32)] * NBUF
            + [pltpu.SemaphoreType.DMA] * (2 * NBUF)
        ),
    )
    def agg_kernel(y_hbm, src_hbm, dst_hbm, out_hbm, sidx_v, didx_v,
                   acc_sh, *rest):
        rows = rest[:NBUF]
        gsem = rest[NBUF:2 * NBUF]
        ssem = rest[2 * NBUF:]
        c = lax.axis_index("c")
        s = lax.axis_index("s")
        wid = c * NS + s
        pltpu.sync_copy(src_hbm.at[wid], sidx_v)
        pltpu.sync_copy(dst_hbm.at[wid], didx_v)
        _fill2(rows[0], 0.0, jnp.float32)
        for k in range(rpt // CB):
            pltpu.sync_copy(rows[0], acc_sh.at[pl.ds(s * rpt + k * CB, CB)])

        for b in range(NBUF):
            pltpu.async_copy(y_hbm.at[sidx_v.at[b]], rows[b], gsem[b])
        plsc.subcore_barrier()

        @pl.loop(0, nch // NBUF)
        def _(k):
            for b in range(NBUF):
                i = k * NBUF + b
                pltpu.make_async_copy(
                    y_hbm.at[sidx_v.at[i]], rows[b], gsem[b]).wait()
                pltpu.async_copy(
                    rows[b], acc_sh.at[didx_v.at[i]], ssem[b], add=True)
            for b in range(NBUF):
                i = k * NBUF + b
                pltpu.make_async_copy(
                    rows[b], acc_sh.at[didx_v.at[i]], ssem[b]).wait()

                @pl.when(i + NBUF < nch)
                def _():
                    pltpu.async_copy(
                        y_hbm.at[sidx_v.at[i + NBUF]], rows[b], gsem[b])

        plsc.subcore_barrier()
        pltpu.sync_copy(acc_sh.at[pl.ds(s * rpt, rpt)],
                        out_hbm.at[c, pl.ds(s * rpt, rpt)])

    return agg_kernel


def _make_agg2_kernel(nch2, npad):
    rpt = npad // NS

    @functools.partial(
        pl.kernel,
        out_type=jax.ShapeDtypeStruct((NC, npad), jnp.float32),
        mesh=_mesh(),
        compiler_params=pltpu.CompilerParams(needs_layout_passes=False,
                                            use_tc_tiling_on_sc=False),
        scratch_types=[
            pltpu.VMEM((nch2, CB), jnp.int32),
            pltpu.VMEM((nch2, CB), jnp.int32),
            pltpu.VMEM((npad,), jnp.float32),
            pltpu.VMEM((npad,), jnp.float32),
            pltpu.VMEM((nch2, CB), jnp.float32),
            pltpu.VMEM((CB,), jnp.float32),
            pltpu.VMEM((16,), jnp.float32),
            pltpu.VMEM((rpt,), jnp.float32),
            pltpu.VMEM((rpt,), jnp.float32),
            pltpu.VMEM_SHARED((npad,), jnp.float32),
            pltpu.SemaphoreType.DMA,
        ],
    )
    def agg2_kernel(t0_hbm, t1_hbm, dinv_hbm, b2_hbm, src_hbm, dst_hbm,
                    out_hbm, sidx_v, didx_v, t_v, dinv_v, vals_v, zero_v,
                    b2_v, acc_v, outb_v, acc_sh, q):
        c = lax.axis_index("c")
        s = lax.axis_index("s")
        pltpu.sync_copy(src_hbm.at[s], sidx_v)
        pltpu.sync_copy(dst_hbm.at[s], didx_v)

        @pl.when(c == 0)
        def _():
            pltpu.sync_copy(t0_hbm, t_v)

        @pl.when(c != 0)
        def _():
            pltpu.sync_copy(t1_hbm, t_v)

        pltpu.sync_copy(dinv_hbm, dinv_v)
        pltpu.sync_copy(b2_hbm.at[c], b2_v)
        _fill(zero_v, 0.0, jnp.float32)
        for k in range(rpt // CB):
            pltpu.sync_copy(zero_v, acc_sh.at[pl.ds(s * rpt + k * CB, CB)])

        @pl.loop(0, nch2)
        def _(i):
            for j in range(CB // 16):
                sreg = sidx_v[i, pl.ds(j * 16, 16)]
                vals_v[i, pl.ds(j * 16, 16)] = plsc.load_gather(t_v, [sreg])

        plsc.subcore_barrier()

        @pl.loop(0, nch2)
        def _(i):
            pltpu.async_copy(vals_v.at[i], acc_sh.at[didx_v.at[i]], q,
                             add=True)

        @pl.loop(0, nch2)
        def _(i):
            pltpu.make_async_copy(vals_v.at[i], acc_sh.at[didx_v.at[i]],
                                  q).wait()

        plsc.subcore_barrier()
        pltpu.sync_copy(acc_sh.at[pl.ds(s * rpt, rpt)], acc_v)
        breg = b2_v[pl.ds(0, 16)]

        @pl.loop(0, rpt // 16)
        def _(j):
            off = s * rpt + j * 16
            agg = acc_v[pl.ds(j * 16, 16)] + t_v[pl.ds(off, 16)]
            outb_v[pl.ds(j * 16, 16)] = (dinv_v[pl.ds(off, 16)] * agg + breg)

        pltpu.sync_copy(outb_v, out_hbm.at[c, pl.ds(s * rpt, rpt)])

    return agg2_kernel


def _tcxw(x, w1):
    n, d_in = x.shape
    d_hid = w1.shape[1]
    blk = 1000 if n % 1000 == 0 else n
    assert n % blk == 0

    def body(x_ref, w1_ref, xw_ref):
        xw_ref[...] = jnp.dot(x_ref[...], w1_ref[...],
                              preferred_element_type=jnp.float32)

    return pl.pallas_call(
        body,
        grid=(n // blk,),
        in_specs=[
            pl.BlockSpec((blk, d_in), lambda i: (i, 0)),
            pl.BlockSpec((d_in, d_hid), lambda i: (0, 0)),
        ],
        out_specs=pl.BlockSpec((blk, d_hid), lambda i: (i, 0)),
        out_shape=jax.ShapeDtypeStruct((n, d_hid), jnp.float32),
    )(x, w1)


def _tcy(degT, xw):
    n, d_hid = xw.shape
    blk = 1000 if n % 1000 == 0 else n
    assert n % blk == 0

    def body(deg_ref, xw_ref, y_ref, dinv_ref):
        deg = deg_ref[:, 0:1] + deg_ref[:, 1:2] + 1.0
        dinv = lax.rsqrt(deg)
        y_ref[...] = xw_ref[...] * dinv
        dinv_ref[...] = dinv

    return pl.pallas_call(
        body,
        grid=(n // blk,),
        in_specs=[
            pl.BlockSpec((blk, 2), lambda i: (i, 0)),
            pl.BlockSpec((blk, d_hid), lambda i: (i, 0)),
        ],
        out_specs=[
            pl.BlockSpec((blk, d_hid), lambda i: (i, 0)),
            pl.BlockSpec((blk, 1), lambda i: (i, 0)),
        ],
        out_shape=[
            jax.ShapeDtypeStruct((n, d_hid), jnp.float32),
            jax.ShapeDtypeStruct((n, 1), jnp.float32),
        ],
    )(degT, xw)


def _tc2(aggp, y0, dinv, w1, b1, w2):
    n, d_in = y0.shape
    d_hid = w1.shape[1]
    d_out = w2.shape[1]
    blk = 1000 if n % 1000 == 0 else n
    assert n % blk == 0

    def body(a0_ref, a1_ref, y0_ref, dinv_ref, w1_ref, b1_ref, w2_ref,
             y2_ref):
        dinv = dinv_ref[...]
        a = (a0_ref[0] + a1_ref[0] + y0_ref[...]) * dinv
        aw = jnp.dot(a, w1_ref[...], preferred_element_type=jnp.float32)
        h = jnp.maximum(aw + b1_ref[...], 0.0)
        hw = jnp.dot(h, w2_ref[...], preferred_element_type=jnp.float32)
        y2_ref[...] = hw * dinv

    return pl.pallas_call(
        body,
        grid=(n // blk,),
        in_specs=[
            pl.BlockSpec((1, blk, d_in), lambda i: (0, i, 0)),
            pl.BlockSpec((1, blk, d_in), lambda i: (1, i, 0)),
            pl.BlockSpec((blk, d_in), lambda i: (i, 0)),
            pl.BlockSpec((blk, 1), lambda i: (i, 0)),
            pl.BlockSpec((d_in, d_hid), lambda i: (0, 0)),
            pl.BlockSpec((1, d_hid), lambda i: (0, 0)),
            pl.BlockSpec((d_hid, d_out), lambda i: (0, 0)),
        ],
        out_specs=pl.BlockSpec((blk, d_out), lambda i: (i, 0)),
        out_shape=jax.ShapeDtypeStruct((n, d_out), jnp.float32),
    )(aggp, aggp, y0, dinv, w1, b1, w2)


def kernel(x, edge_index, W1, b1, W2, b2):
    n, _ = x.shape
    e = edge_index.shape[1]
    npad = NS * _rows_per_tile(n)
    assert e % (NC * NS) == 0 and npad > n
    ept = e // (NC * NS)
    nch = (ept + NBUF * CB - 1) // (NBUF * CB) * NBUF
    eptp = nch * CB

    src = edge_index[0]
    dst = edge_index[1]
    pad = ((0, 0), (0, eptp - ept))
    src3 = jnp.pad(src.reshape(NC * NS, ept), pad).reshape(NC * NS, nch, CB)
    dst3 = jnp.pad(dst.reshape(NC * NS, ept), pad,
                   constant_values=npad - 1).reshape(NC * NS, nch, CB)

    degp = _make_degree_kernel(nch, npad)(dst3)
    degT = jnp.transpose(degp[:, :n])
    y0, dinv = _tcy(degT, x)
    aggp = _make_agg_kernel(nch, npad, x.shape[1])(y0, src3, dst3)
    y2 = _tc2(aggp, y0, dinv, W1, b1.reshape(1, -1), W2)

    tpad = (0, npad - n)
    t0p = jnp.pad(y2[:, 0], tpad)
    t1p = jnp.pad(y2[:, 1], tpad)
    dinvp = jnp.pad(dinv[:, 0], tpad)
    b2b = jnp.broadcast_to(b2.reshape(-1, 1), (2, 16))
    src16 = src3.reshape(NS, NC * nch, CB)
    dst16 = dst3.reshape(NS, NC * nch, CB)
    planes = _make_agg2_kernel(NC * nch, npad)(
        t0p, t1p, dinvp, b2b, src16, dst16)
    return jnp.stack([planes[0, :n], planes[1, :n]], axis=1)


# speedup vs baseline: 1.3677x; 1.3677x over previous
import functools

import jax
import jax.numpy as jnp
from jax import lax
from jax.experimental import pallas as pl
from jax.experimental.pallas import tpu as pltpu
from jax.experimental.pallas import tpu_sc as plsc

NC = 2
NS = 16
CB = 32
NBUF = 5


def _mesh():
    return plsc.VectorSubcoreMesh(core_axis_name="c", subcore_axis_name="s",
                                  num_cores=NC, num_subcores=NS)


def _rows_per_tile(n):
    return ((n + NS - 1) // NS + CB - 1) // CB * CB


def _fill(ref, value, dtype):
    (n,) = ref.shape

    @pl.loop(0, n // 16)
    def _(j):
        ref[pl.ds(j * 16, 16)] = jnp.full((16,), value, dtype)


def _fill2(ref, value, dtype):
    rows, cols = ref.shape

    @pl.loop(0, rows)
    def _(r):
        @pl.loop(0, cols // 16)
        def _(j):
            ref[r, pl.ds(j * 16, 16)] = jnp.full((16,), value, dtype)


def _make_degree_kernel(nch, npad):
    rpt = npad // NS

    @functools.partial(
        pl.kernel,
        out_type=jax.ShapeDtypeStruct((NC, npad), jnp.float32),
        mesh=_mesh(),
        scratch_types=[
            pltpu.VMEM((nch, CB), jnp.int32),
            pltpu.VMEM((CB,), jnp.float32),
            pltpu.VMEM((CB,), jnp.float32),
            pltpu.VMEM_SHARED((npad,), jnp.float32),
            pltpu.SemaphoreType.DMA,
        ],
    )
    def deg_kernel(dst_hbm, out_hbm, didx_v, ones_v, zero_v, acc_sh, sem):
        c = lax.axis_index("c")
        s = lax.axis_index("s")
        wid = c * NS + s
        pltpu.sync_copy(dst_hbm.at[wid], didx_v)
        _fill(ones_v, 1.0, jnp.float32)
        _fill(zero_v, 0.0, jnp.float32)
        for k in range(rpt // CB):
            pltpu.sync_copy(zero_v, acc_sh.at[pl.ds(s * rpt + k * CB, CB)])
        plsc.subcore_barrier()

        @pl.loop(0, nch)
        def _(i):
            pltpu.async_copy(ones_v, acc_sh.at[didx_v.at[i]], sem, add=True)

        @pl.loop(0, nch)
        def _(i):
            pltpu.make_async_copy(ones_v, acc_sh.at[didx_v.at[i]], sem).wait()

        plsc.subcore_barrier()
        pltpu.sync_copy(acc_sh.at[pl.ds(s * rpt, rpt)],
                        out_hbm.at[c, pl.ds(s * rpt, rpt)])

    return deg_kernel


def _make_agg_kernel(nch, npad, d):
    assert nch % NBUF == 0 and nch // NBUF >= 2
    rpt = npad // NS

    @functools.partial(
        pl.kernel,
        out_type=jax.ShapeDtypeStruct((NC, npad, d), jnp.float32),
        mesh=_mesh(),
        compiler_params=pltpu.CompilerParams(use_tc_tiling_on_sc=False),
        scratch_types=(
            [pltpu.VMEM((nch, CB), jnp.int32)] * 2
            + [pltpu.VMEM_SHARED((npad, d), jnp.float32)]
            + [pltpu.VMEM((CB, d), jnp.float32)] * NBUF
            + [pltpu.SemaphoreType.DMA] * (2 * NBUF)
        ),
    )
    def agg_kernel(y_hbm, src_hbm, dst_hbm, out_hbm, sidx_v, didx_v,
                   acc_sh, *rest):
        rows = rest[:NBUF]
        gsem = rest[NBUF:2 * NBUF]
        ssem = rest[2 * NBUF:]
        c = lax.axis_index("c")
        s = lax.axis_index("s")
        wid = c * NS + s
        pltpu.sync_copy(src_hbm.at[wid], sidx_v)
        pltpu.sync_copy(dst_hbm.at[wid], didx_v)
        _fill2(rows[0], 0.0, jnp.float32)
        for k in range(rpt // CB):
            pltpu.sync_copy(rows[0], acc_sh.at[pl.ds(s * rpt + k * CB, CB)])

        for b in range(NBUF):
            pltpu.async_copy(y_hbm.at[sidx_v.at[b]], rows[b], gsem[b])
        plsc.subcore_barrier()

        @pl.loop(0, nch // NBUF)
        def _(k):
            for b in range(NBUF):
                i = k * NBUF + b
                pltpu.make_async_copy(
                    y_hbm.at[sidx_v.at[i]], rows[b], gsem[b]).wait()
                pltpu.async_copy(
                    rows[b], acc_sh.at[didx_v.at[i]], ssem[b], add=True)
            for b in range(NBUF):
                i = k * NBUF + b
                pltpu.make_async_copy(
                    rows[b], acc_sh.at[didx_v.at[i]], ssem[b]).wait()

                @pl.when(i + NBUF < nch)
                def _():
                    pltpu.async_copy(
                        y_hbm.at[sidx_v.at[i + NBUF]], rows[b], gsem[b])

        plsc.subcore_barrier()
        pltpu.sync_copy(acc_sh.at[pl.ds(s * rpt, rpt)],
                        out_hbm.at[c, pl.ds(s * rpt, rpt)])

    return agg_kernel


def _make_agg2_kernel(nch2, npad):
    rpt = npad // NS

    @functools.partial(
        pl.kernel,
        out_type=jax.ShapeDtypeStruct((NC, npad), jnp.float32),
        mesh=_mesh(),
        compiler_params=pltpu.CompilerParams(needs_layout_passes=False,
                                            use_tc_tiling_on_sc=False),
        scratch_types=[
            pltpu.VMEM((nch2, CB), jnp.int32),
            pltpu.VMEM((nch2, CB), jnp.int32),
            pltpu.VMEM((npad,), jnp.float32),
            pltpu.VMEM((npad,), jnp.float32),
            pltpu.VMEM((nch2, CB), jnp.float32),
            pltpu.VMEM((CB,), jnp.float32),
            pltpu.VMEM((16,), jnp.float32),
            pltpu.VMEM((rpt,), jnp.float32),
            pltpu.VMEM((rpt,), jnp.float32),
            pltpu.VMEM_SHARED((npad,), jnp.float32),
            pltpu.SemaphoreType.DMA,
        ],
    )
    def agg2_kernel(t0_hbm, t1_hbm, dinv_hbm, b2_hbm, src_hbm, dst_hbm,
                    out_hbm, sidx_v, didx_v, t_v, dinv_v, vals_v, zero_v,
                    b2_v, acc_v, outb_v, acc_sh, q):
        c = lax.axis_index("c")
        s = lax.axis_index("s")
        pltpu.sync_copy(src_hbm.at[s], sidx_v)
        pltpu.sync_copy(dst_hbm.at[s], didx_v)

        @pl.when(c == 0)
        def _():
            pltpu.sync_copy(t0_hbm, t_v)

        @pl.when(c != 0)
        def _():
            pltpu.sync_copy(t1_hbm, t_v)

        pltpu.sync_copy(dinv_hbm, dinv_v)
        pltpu.sync_copy(b2_hbm.at[c], b2_v)
        _fill(zero_v, 0.0, jnp.float32)
        for k in range(rpt // CB):
            pltpu.sync_copy(zero_v, acc_sh.at[pl.ds(s * rpt + k * CB, CB)])

        @pl.loop(0, nch2)
        def _(i):
            for j in range(CB // 16):
                sreg = sidx_v[i, pl.ds(j * 16, 16)]
                vals_v[i, pl.ds(j * 16, 16)] = plsc.load_gather(t_v, [sreg])

        plsc.subcore_barrier()

        @pl.loop(0, nch2)
        def _(i):
            pltpu.async_copy(vals_v.at[i], acc_sh.at[didx_v.at[i]], q,
                             add=True)

        @pl.loop(0, nch2)
        def _(i):
            pltpu.make_async_copy(vals_v.at[i], acc_sh.at[didx_v.at[i]],
                                  q).wait()

        plsc.subcore_barrier()
        pltpu.sync_copy(acc_sh.at[pl.ds(s * rpt, rpt)], acc_v)
        breg = b2_v[pl.ds(0, 16)]

        @pl.loop(0, rpt // 16)
        def _(j):
            off = s * rpt + j * 16
            agg = acc_v[pl.ds(j * 16, 16)] + t_v[pl.ds(off, 16)]
            outb_v[pl.ds(j * 16, 16)] = (dinv_v[pl.ds(off, 16)] * agg + breg)

        pltpu.sync_copy(outb_v, out_hbm.at[c, pl.ds(s * rpt, rpt)])

    return agg2_kernel


def _tcy(degT, xw):
    n, d_hid = xw.shape
    blk = 1000 if n % 1000 == 0 else n
    assert n % blk == 0

    def body(deg_ref, xw_ref, y_ref, dinv_ref):
        deg = deg_ref[:, 0:1] + deg_ref[:, 1:2] + 1.0
        dinv = lax.rsqrt(deg)
        y_ref[...] = xw_ref[...] * dinv
        dinv_ref[...] = dinv

    return pl.pallas_call(
        body,
        grid=(n // blk,),
        in_specs=[
            pl.BlockSpec((blk, 2), lambda i: (i, 0)),
            pl.BlockSpec((blk, d_hid), lambda i: (i, 0)),
        ],
        out_specs=[
            pl.BlockSpec((blk, d_hid), lambda i: (i, 0)),
            pl.BlockSpec((blk, 1), lambda i: (i, 0)),
        ],
        out_shape=[
            jax.ShapeDtypeStruct((n, d_hid), jnp.float32),
            jax.ShapeDtypeStruct((n, 1), jnp.float32),
        ],
    )(degT, xw)


def _tc2(aggp, y0, dinv, w1, b1, w2):
    n, d_in = y0.shape
    d_hid = w1.shape[1]
    d_out = w2.shape[1]
    blk = 1000 if n % 1000 == 0 else n
    assert n % blk == 0

    def body(a0_ref, a1_ref, y0_ref, dinv_ref, w1_ref, b1_ref, w2_ref,
             y2_ref):
        dinv = dinv_ref[...]
        a = (a0_ref[0] + a1_ref[0] + y0_ref[...]) * dinv
        aw = jnp.dot(a, w1_ref[...], preferred_element_type=jnp.float32)
        h = jnp.maximum(aw + b1_ref[...], 0.0)
        hw = jnp.dot(h, w2_ref[...], preferred_element_type=jnp.float32)
        y2_ref[...] = hw * dinv

    return pl.pallas_call(
        body,
        grid=(n // blk,),
        in_specs=[
            pl.BlockSpec((1, blk, d_in), lambda i: (0, i, 0)),
            pl.BlockSpec((1, blk, d_in), lambda i: (1, i, 0)),
            pl.BlockSpec((blk, d_in), lambda i: (i, 0)),
            pl.BlockSpec((blk, 1), lambda i: (i, 0)),
            pl.BlockSpec((d_in, d_hid), lambda i: (0, 0)),
            pl.BlockSpec((1, d_hid), lambda i: (0, 0)),
            pl.BlockSpec((d_hid, d_out), lambda i: (0, 0)),
        ],
        out_specs=pl.BlockSpec((blk, d_out), lambda i: (i, 0)),
        out_shape=jax.ShapeDtypeStruct((n, d_out), jnp.float32),
    )(aggp, aggp, y0, dinv, w1, b1, w2)


def kernel(x, edge_index, W1, b1, W2, b2):
    n, _ = x.shape
    e = edge_index.shape[1]
    npad = NS * _rows_per_tile(n)
    assert e % (NC * NS) == 0 and npad > n
    ept = e // (NC * NS)
    nch = (ept + NBUF * CB - 1) // (NBUF * CB) * NBUF
    eptp = nch * CB

    src = edge_index[0]
    dst = edge_index[1]
    pad = ((0, 0), (0, eptp - ept))
    src3 = jnp.pad(src.reshape(NC * NS, ept), pad).reshape(NC * NS, nch, CB)
    dst3 = jnp.pad(dst.reshape(NC * NS, ept), pad,
                   constant_values=npad - 1).reshape(NC * NS, nch, CB)

    degp = _make_degree_kernel(nch, npad)(dst3)
    degT = jnp.transpose(degp[:, :n])
    y0, dinv = _tcy(degT, x)
    aggp = _make_agg_kernel(nch, npad, x.shape[1])(y0, src3, dst3)
    y2 = _tc2(aggp, y0, dinv, W1, b1.reshape(1, -1), W2)

    tpad = (0, npad - n)
    t0p = jnp.pad(y2[:, 0], tpad)
    t1p = jnp.pad(y2[:, 1], tpad)
    dinvp = jnp.pad(dinv[:, 0], tpad)
    b2b = jnp.broadcast_to(b2.reshape(-1, 1), (2, 16))
    src16 = src3.reshape(NS, NC * nch, CB)
    dst16 = dst3.reshape(NS, NC * nch, CB)
    planes = _make_agg2_kernel(NC * nch, npad)(
        t0p, t1p, dinvp, b2b, src16, dst16)
    return jnp.stack([planes[0, :n], planes[1, :n]], axis=1)
